# baseline (speedup 1.0000x reference)
import functools

import jax
import jax.numpy as jnp
from jax import lax
from jax.experimental import pallas as pl
from jax.experimental.pallas import tpu as pltpu
from jax.experimental.pallas import tpu_sc as plsc

B = 4096
V = 7000
D = 100
DP = 128
M = 32


def _precompute_body(w_ref, mm_ref, out_ref):
    e = jnp.exp(w_ref[...])
    z = jnp.dot(e, mm_ref[...], preferred_element_type=jnp.float32)
    n = jnp.sqrt(jnp.sum(z * z, axis=-1, keepdims=True))
    out_ref[:, :M] = z / jnp.maximum(n, 1e-12)


@functools.lru_cache(maxsize=None)
def _make_tc_precompute():
    blk = 1000
    return pl.pallas_call(
        _precompute_body,
        grid=(V // blk,),
        in_specs=[
            pl.BlockSpec((blk, D), lambda i: (i, 0)),
            pl.BlockSpec((D, M), lambda i: (0, 0)),
        ],
        out_specs=pl.BlockSpec((blk, DP), lambda i: (i, 0)),
        out_shape=jax.ShapeDtypeStruct((V, DP), jnp.float32),
    )


@functools.lru_cache(maxsize=None)
def _make_sc_gather_t():
    info = plsc.get_sparse_core_info()
    nw = info.num_cores * info.num_subcores
    b_per_w = B // nw
    mesh = plsc.VectorSubcoreMesh(core_axis_name="c", subcore_axis_name="s")

    @functools.partial(
        pl.kernel,
        mesh=mesh,
        out_type=jax.ShapeDtypeStruct((M, B), jnp.float32),
        scratch_types=[
            pltpu.VMEM((b_per_w,), jnp.int32),
            pltpu.VMEM((b_per_w, DP), jnp.float32),
            pltpu.VMEM((M, b_per_w), jnp.float32),
            pltpu.SemaphoreType.DMA,
        ],
        compiler_params=pltpu.CompilerParams(needs_layout_passes=False),
    )
    def gather_k(idx_hbm, table_hbm, out_hbm, idx_v, rows_v, tbuf, sem):
        wid = lax.axis_index("s") * info.num_cores + lax.axis_index("c")
        base = wid * b_per_w
        pltpu.sync_copy(idx_hbm.at[pl.ds(base, b_per_w)], idx_v)
        pltpu.async_copy(table_hbm.at[idx_v], rows_v, sem).wait()
        ji = lax.iota(jnp.int32, 16)
        for c in range(M):
            cv = jnp.full((16,), c, jnp.int32)
            for j0 in range(0, b_per_w, 16):
                vals = plsc.load_gather(rows_v, [ji + j0, cv])
                plsc.store_scatter(tbuf, [cv, ji + j0], vals)
        pltpu.sync_copy(tbuf, out_hbm.at[:, pl.ds(base, b_per_w)])

    return gather_k


def kernel(idx, weight_embedding, main_modes):
    pn = _make_tc_precompute()(weight_embedding, main_modes)
    out_t = _make_sc_gather_t()(idx.astype(jnp.int32), pn)
    return jnp.transpose(out_t)[:, None, :]
